# Initial kernel scaffold; baseline (speedup 1.0000x reference)
import jax
import jax.numpy as jnp
from jax.experimental import pallas as pl


def kernel(x, W1, b1, W2, b2, W3, b3, thr, Wm, bm, Wo, bo):
    raise NotImplementedError("write your pallas kernel here")


# speedup vs baseline: 2.6121x; 2.6121x over previous
import jax
import jax.numpy as jnp
from jax import lax
from jax.experimental import pallas as pl
from jax.experimental.pallas import tpu as pltpu

_CAND_R = 2
_TEMP = 0.1
_MAX_EDGES = 8
_MIN_EDGES = 3

_H = 100
_W = 100
_WP = 104
_HP = 108
_NROWS = _HP * _WP
_BASE = 2 * _WP + 2
_NSTEPS = 12
_SBB = 7
_NB = _SBB * 128
_NP = _NSTEPS * _NB


def _stencil():
    out = []
    for dy in range(-_CAND_R, _CAND_R + 1):
        for dx in range(-_CAND_R, _CAND_R + 1):
            if dy == 0 and dx == 0:
                continue
            out.append((dy, dx, dx / _CAND_R, dy / _CAND_R))
    return out


def _graph_kernel(gf_ref, w1s_ref, w1t_ref, w1rt_ref, b1_ref, w2_ref, b2_ref,
                  w3_ref, b3_ref, thr_ref, wm_ref, bm_ref, wo_ref, bo_ref,
                  out_ref, t1_scr, m_scr):
    f32 = jnp.float32
    dn = (((1,), (1,)), ((), ()))
    i = pl.program_id(0)

    @pl.when(i == 0)
    def _precompute():
        gf = gf_ref[...]
        t1_scr[...] = lax.dot_general(gf, w1t_ref[...], dn,
                                      preferred_element_type=f32)
        m_scr[...] = jnp.maximum(
            lax.dot_general(gf, wm_ref[...], dn, preferred_element_type=f32)
            + bm_ref[...], 0.0)

    base_i = i * _NB
    slab0 = gf_ref[pl.ds(_BASE + base_i, _NB), :]
    s1 = lax.dot_general(slab0, w1s_ref[...], dn, preferred_element_type=f32)

    w1rt = w1rt_ref[...]
    b1 = b1_ref[...]
    w2 = w2_ref[...]
    b2 = b2_ref[...]
    w3 = w3_ref[...].reshape(1, 1, 32)
    b3 = b3_ref[0, 0]
    thr_val = jax.nn.sigmoid(thr_ref[0, 0])

    stencil = _stencil()
    scores = []
    for (dy, dx, rx, ry) in stencil:
        start = _BASE + dy * _WP + dx
        tsh = t1_scr[pl.ds(start + base_i, _NB), :]
        r1k = rx * w1rt[0:1, :] + ry * w1rt[1:2, :] + b1
        h1 = jnp.maximum(s1 + tsh + r1k, 0.0)
        h2 = jnp.maximum(
            lax.dot_general(h1, w2, dn, preferred_element_type=f32) + b2, 0.0)
        z3 = jnp.sum(h2.reshape(_SBB, 128, 32) * w3, axis=2) + b3
        scores.append(jax.nn.sigmoid(z3))

    cnt = jnp.zeros((_SBB, 128), f32)
    for s in scores:
        cnt = cnt + (s >= thr_val).astype(f32)
    use_max = cnt > float(_MAX_EDGES)
    use_min = cnt < float(_MIN_EDGES)
    use_thr = jnp.logical_and(jnp.logical_not(use_max),
                              jnp.logical_not(use_min))

    ranks = [jnp.zeros((_SBB, 128), f32) for _ in range(len(scores))]
    for a in range(len(scores)):
        for b in range(a + 1, len(scores)):
            ranks[b] = ranks[b] + (scores[a] >= scores[b]).astype(f32)
            ranks[a] = ranks[a] + (scores[b] > scores[a]).astype(f32)

    wts = []
    wsum = jnp.zeros((_SBB, 128), f32)
    for k in range(len(scores)):
        s = scores[k]
        top_max = ranks[k] < float(_MAX_EDGES)
        top_min = ranks[k] < float(_MIN_EDGES)
        m_thr = s >= thr_val
        fin = jnp.logical_or(
            jnp.logical_or(jnp.logical_and(use_max, top_max),
                           jnp.logical_and(use_min, top_min)),
            jnp.logical_and(use_thr, m_thr))
        keep = jax.nn.sigmoid((s - thr_val) / _TEMP)
        w = s * keep * fin.astype(f32)
        wts.append(w)
        wsum = wsum + w

    agg = jnp.zeros((_SBB, 128, 128), f32)
    for k, (dy, dx, rx, ry) in enumerate(stencil):
        start = _BASE + dy * _WP + dx
        msh = m_scr[pl.ds(start + base_i, _NB), :].reshape(_SBB, 128, 128)
        agg = agg + msh * wts[k][:, :, None]
    agg = agg / (wsum[:, :, None] + 1e-6)
    aggregated = agg.reshape(_NB, 128)

    out_ref[...] = slab0 + lax.dot_general(
        aggregated, wo_ref[...], dn, preferred_element_type=f32) + bo_ref[...]


def kernel(x, W1, b1, W2, b2, W3, b3, thr, Wm, bm, Wo, bo):
    B, C, H, W = x.shape
    xt = jnp.transpose(x[0], (1, 2, 0))
    xp = jnp.pad(xt, ((2, 6), (2, 2), (0, 0)), mode='edge')
    gf = xp.reshape(_NROWS, C)

    W1s = W1[:, :C]
    W1t = W1[:, C:2 * C]
    W1rT = W1[:, 2 * C:].T

    full = lambda shape: pl.BlockSpec(shape, lambda i: (0, 0))
    out_slab = pl.pallas_call(
        _graph_kernel,
        grid=(_NSTEPS,),
        in_specs=[
            full((_NROWS, C)),
            full((64, C)), full((64, C)), full((2, 64)), full((1, 64)),
            full((32, 64)), full((1, 32)),
            full((1, 32)), full((1, 1)), full((1, 1)),
            full((C, C)), full((1, C)),
            full((C, C)), full((1, C)),
        ],
        out_specs=pl.BlockSpec((_NB, C), lambda i: (i, 0)),
        out_shape=jax.ShapeDtypeStruct((_NP, C), jnp.float32),
        scratch_shapes=[
            pltpu.VMEM((_NROWS, 64), jnp.float32),
            pltpu.VMEM((_NROWS, C), jnp.float32),
        ],
    )(gf, W1s, W1t, W1rT, b1.reshape(1, -1), W2, b2.reshape(1, -1),
      W3, b3.reshape(1, 1), thr.reshape(1, 1), Wm, bm.reshape(1, -1),
      Wo, bo.reshape(1, -1))

    out = out_slab[:_H * _WP].reshape(_H, _WP, C)[:, :_W, :]
    return jnp.transpose(out, (2, 0, 1))[None]


# speedup vs baseline: 2.9930x; 1.1458x over previous
import jax
import jax.numpy as jnp
from jax import lax
from jax.experimental import pallas as pl
from jax.experimental.pallas import tpu as pltpu

_CAND_R = 2
_TEMP = 0.1
_MAX_EDGES = 8
_MIN_EDGES = 3
_K = 24

_H = 100
_W = 100
_WP = 104
_HP = 115
_NROWS = _HP * _WP
_BASE = 2 * _WP + 2
_NSTEPS = 11
_SBB = 8
_NB = _SBB * 128
_NP = _NSTEPS * _NB


def _stencil():
    out = []
    for dy in range(-_CAND_R, _CAND_R + 1):
        for dx in range(-_CAND_R, _CAND_R + 1):
            if dy == 0 and dx == 0:
                continue
            out.append((dy, dx, dx / _CAND_R, dy / _CAND_R))
    return out


def _graph_kernel(gf_ref, w1s_ref, w1t_ref, w1rt_ref, b1_ref, w2_ref, b2_ref,
                  w3_ref, b3_ref, thr_ref, wm_ref, bm_ref, wo_ref, bo_ref,
                  out_ref, t1_scr, m_scr, h1_scr):
    f32 = jnp.float32
    dn = (((1,), (1,)), ((), ()))
    i = pl.program_id(0)

    @pl.when(i == 0)
    def _precompute():
        gf = gf_ref[...]
        t1_scr[...] = lax.dot_general(gf, w1t_ref[...], dn,
                                      preferred_element_type=f32)
        m_scr[...] = jnp.maximum(
            lax.dot_general(gf, wm_ref[...], dn, preferred_element_type=f32)
            + bm_ref[...], 0.0)

    base_i = i * _NB
    slab0 = gf_ref[pl.ds(_BASE + base_i, _NB), :]
    s1 = lax.dot_general(slab0, w1s_ref[...], dn, preferred_element_type=f32)

    w1rt = w1rt_ref[...]
    b1 = b1_ref[...]
    w3 = w3_ref[...].reshape(1, 1, 32)
    b3 = b3_ref[0, 0]
    thr_val = jax.nn.sigmoid(thr_ref[0, 0])

    stencil = _stencil()
    for k, (dy, dx, rx, ry) in enumerate(stencil):
        start = _BASE + dy * _WP + dx
        tsh = t1_scr[pl.ds(start + base_i, _NB), :]
        r1k = rx * w1rt[0:1, :] + ry * w1rt[1:2, :] + b1
        h1_scr[pl.ds(k * _NB, _NB), :] = jnp.maximum(s1 + tsh + r1k, 0.0)

    h2 = jnp.maximum(
        lax.dot_general(h1_scr[...], w2_ref[...], dn,
                        preferred_element_type=f32) + b2_ref[...], 0.0)
    z3 = jnp.sum(h2.reshape(_K * _SBB, 128, 32) * w3, axis=2) + b3
    S = jax.nn.sigmoid(z3)
    keepS = jax.nn.sigmoid((S - thr_val) * (1.0 / _TEMP))
    cntf = (S >= thr_val).astype(f32)

    sc = [S[k * _SBB:(k + 1) * _SBB] for k in range(_K)]
    cnt = cntf[0:_SBB]
    for k in range(1, _K):
        cnt = cnt + cntf[k * _SBB:(k + 1) * _SBB]
    use_max = cnt > float(_MAX_EDGES)
    use_min = cnt < float(_MIN_EDGES)
    use_thr = jnp.logical_and(jnp.logical_not(use_max),
                              jnp.logical_not(use_min))

    acc = [jnp.zeros((_SBB, 128), f32) for _ in range(_K)]
    for a in range(_K):
        for b in range(a + 1, _K):
            gef = (sc[a] >= sc[b]).astype(f32)
            acc[a] = acc[a] - gef
            acc[b] = acc[b] + gef

    wts = []
    wsum = jnp.zeros((_SBB, 128), f32)
    for k in range(_K):
        rank = acc[k] + float(_K - 1 - k)
        top_max = rank < float(_MAX_EDGES)
        top_min = rank < float(_MIN_EDGES)
        m_thr = cntf[k * _SBB:(k + 1) * _SBB] > 0.0
        fin = jnp.logical_or(
            jnp.logical_or(jnp.logical_and(use_max, top_max),
                           jnp.logical_and(use_min, top_min)),
            jnp.logical_and(use_thr, m_thr))
        w = sc[k] * keepS[k * _SBB:(k + 1) * _SBB] * fin.astype(f32)
        wts.append(w)
        wsum = wsum + w

    agg = jnp.zeros((_SBB, 128, 128), f32)
    for k, (dy, dx, rx, ry) in enumerate(stencil):
        start = _BASE + dy * _WP + dx
        msh = m_scr[pl.ds(start + base_i, _NB), :].reshape(_SBB, 128, 128)
        agg = agg + msh * wts[k][:, :, None]
    agg = agg / (wsum[:, :, None] + 1e-6)
    aggregated = agg.reshape(_NB, 128)

    out_ref[...] = slab0 + lax.dot_general(
        aggregated, wo_ref[...], dn, preferred_element_type=f32) + bo_ref[...]


def kernel(x, W1, b1, W2, b2, W3, b3, thr, Wm, bm, Wo, bo):
    B, C, H, W = x.shape
    xt = jnp.transpose(x[0], (1, 2, 0))
    xp = jnp.pad(xt, ((2, 13), (2, 2), (0, 0)), mode='edge')
    gf = xp.reshape(_NROWS, C)

    W1s = W1[:, :C]
    W1t = W1[:, C:2 * C]
    W1rT = W1[:, 2 * C:].T

    full = lambda shape: pl.BlockSpec(shape, lambda i: (0, 0))
    out_slab = pl.pallas_call(
        _graph_kernel,
        grid=(_NSTEPS,),
        in_specs=[
            full((_NROWS, C)),
            full((64, C)), full((64, C)), full((2, 64)), full((1, 64)),
            full((32, 64)), full((1, 32)),
            full((1, 32)), full((1, 1)), full((1, 1)),
            full((C, C)), full((1, C)),
            full((C, C)), full((1, C)),
        ],
        out_specs=pl.BlockSpec((_NB, C), lambda i: (i, 0)),
        out_shape=jax.ShapeDtypeStruct((_NP, C), jnp.float32),
        scratch_shapes=[
            pltpu.VMEM((_NROWS, 64), jnp.float32),
            pltpu.VMEM((_NROWS, C), jnp.float32),
            pltpu.VMEM((_K * _NB, 64), jnp.float32),
        ],
    )(gf, W1s, W1t, W1rT, b1.reshape(1, -1), W2, b2.reshape(1, -1),
      W3, b3.reshape(1, 1), thr.reshape(1, 1), Wm, bm.reshape(1, -1),
      Wo, bo.reshape(1, -1))

    out = out_slab[:_H * _WP].reshape(_H, _WP, C)[:, :_W, :]
    return jnp.transpose(out, (2, 0, 1))[None]


# speedup vs baseline: 11.2917x; 3.7727x over previous
import jax
import jax.numpy as jnp
from jax import lax
from jax.experimental import pallas as pl
from jax.experimental.pallas import tpu as pltpu

_CAND_R = 2
_TEMP = 0.1
_MAX_EDGES = 8
_MIN_EDGES = 3
_K = 24

_H = 100
_W = 100
_WP = 104
_HP = 115
_NROWS = _HP * _WP
_BASE = 2 * _WP + 2
_NSTEPS = 11
_SBB = 8
_NB = _SBB * 128
_NP = _NSTEPS * _NB


def _stencil():
    out = []
    for dy in range(-_CAND_R, _CAND_R + 1):
        for dx in range(-_CAND_R, _CAND_R + 1):
            if dy == 0 and dx == 0:
                continue
            out.append((dy, dx, dx / _CAND_R, dy / _CAND_R))
    return out


def _sigmoid(x):
    return 1.0 / (1.0 + jnp.exp2(x * (-1.4426950408889634)))


def _graph_kernel(gf_ref, w1s_ref, w1t_ref, w1rt_ref, b1_ref, w2_ref, b2_ref,
                  w3_ref, b3_ref, thr_ref, wm_ref, bm_ref, wo_ref, bo_ref,
                  out_ref, t1_scr, m_scr, h1_scr):
    f32 = jnp.float32
    dn = (((1,), (1,)), ((), ()))
    i = pl.program_id(0)

    @pl.when(i == 0)
    def _precompute():
        gf = gf_ref[...]
        t1_scr[...] = lax.dot_general(gf, w1t_ref[...], dn,
                                      preferred_element_type=f32)
        m_scr[...] = jnp.maximum(
            lax.dot_general(gf, wm_ref[...], dn, preferred_element_type=f32)
            + bm_ref[...], 0.0)

    base_i = i * _NB
    slab0 = gf_ref[pl.ds(_BASE + base_i, _NB), :]
    s1 = lax.dot_general(slab0, w1s_ref[...], dn, preferred_element_type=f32)

    w1rt = w1rt_ref[...]
    b1 = b1_ref[...]
    w3 = w3_ref[...].reshape(1, 1, 32)
    b3 = b3_ref[0, 0]
    thr_val = jax.nn.sigmoid(thr_ref[0, 0])

    stencil = _stencil()
    for k, (dy, dx, rx, ry) in enumerate(stencil):
        start = _BASE + dy * _WP + dx
        tsh = t1_scr[pl.ds(start + base_i, _NB), :]
        r1k = rx * w1rt[0:1, :] + ry * w1rt[1:2, :] + b1
        h1_scr[pl.ds(k * _NB, _NB), :] = jnp.maximum(s1 + tsh + r1k, 0.0)

    b2 = b2_ref[...]
    z3_chunks = []
    for c in range(0, _K, 6):
        h1c = h1_scr[pl.ds(c * _NB, 6 * _NB), :]
        h2c = jnp.maximum(
            lax.dot_general(h1c, w2_ref[...], dn,
                            preferred_element_type=f32) + b2, 0.0)
        z3_chunks.append(
            jnp.sum(h2c.reshape(6 * _SBB, 128, 32) * w3, axis=2))
    z3 = jnp.concatenate(z3_chunks, axis=0) + b3

    S3 = _sigmoid(z3).reshape(_K, _SBB, 128)
    keep3 = _sigmoid((S3 - thr_val) * (1.0 / _TEMP))
    mthr3 = S3 >= thr_val
    cnt = jnp.sum(mthr3.astype(f32), axis=0)
    use_max = cnt > float(_MAX_EDGES)
    use_min = cnt < float(_MIN_EDGES)
    use_thr = jnp.logical_and(jnp.logical_not(use_max),
                              jnp.logical_not(use_min))

    rank3 = jnp.zeros((_K, _SBB, 128), f32)
    for kp in range(_K):
        skp = S3[kp:kp + 1]
        gtb = skp > S3
        if kp < _K - 1:
            cmb = jnp.concatenate([gtb[:kp + 1], (skp >= S3)[kp + 1:]],
                                  axis=0)
        else:
            cmb = gtb
        rank3 = rank3 + cmb.astype(f32)

    fin3 = jnp.logical_or(
        jnp.logical_or(
            jnp.logical_and(use_max[None], rank3 < float(_MAX_EDGES)),
            jnp.logical_and(use_min[None], rank3 < float(_MIN_EDGES))),
        jnp.logical_and(use_thr[None], mthr3))
    wts3 = S3 * keep3 * fin3.astype(f32)
    wsum = jnp.sum(wts3, axis=0)

    agg = jnp.zeros((_SBB, 128, 128), f32)
    for k, (dy, dx, rx, ry) in enumerate(stencil):
        start = _BASE + dy * _WP + dx
        msh = m_scr[pl.ds(start + base_i, _NB), :].reshape(_SBB, 128, 128)
        agg = agg + msh * wts3[k][:, :, None]
    agg = agg / (wsum[:, :, None] + 1e-6)
    aggregated = agg.reshape(_NB, 128)

    out_ref[...] = slab0 + lax.dot_general(
        aggregated, wo_ref[...], dn, preferred_element_type=f32) + bo_ref[...]


def kernel(x, W1, b1, W2, b2, W3, b3, thr, Wm, bm, Wo, bo):
    B, C, H, W = x.shape
    xt = jnp.transpose(x[0], (1, 2, 0))
    xp = jnp.pad(xt, ((2, 13), (2, 2), (0, 0)), mode='edge')
    gf = xp.reshape(_NROWS, C)

    W1s = W1[:, :C]
    W1t = W1[:, C:2 * C]
    W1rT = W1[:, 2 * C:].T

    full = lambda shape: pl.BlockSpec(shape, lambda i: (0, 0))
    out_slab = pl.pallas_call(
        _graph_kernel,
        grid=(_NSTEPS,),
        in_specs=[
            full((_NROWS, C)),
            full((64, C)), full((64, C)), full((2, 64)), full((1, 64)),
            full((32, 64)), full((1, 32)),
            full((1, 32)), full((1, 1)), full((1, 1)),
            full((C, C)), full((1, C)),
            full((C, C)), full((1, C)),
        ],
        out_specs=pl.BlockSpec((_NB, C), lambda i: (i, 0)),
        out_shape=jax.ShapeDtypeStruct((_NP, C), jnp.float32),
        scratch_shapes=[
            pltpu.VMEM((_NROWS, 64), jnp.float32),
            pltpu.VMEM((_NROWS, C), jnp.float32),
            pltpu.VMEM((_K * _NB, 64), jnp.float32),
        ],
    )(gf, W1s, W1t, W1rT, b1.reshape(1, -1), W2, b2.reshape(1, -1),
      W3, b3.reshape(1, 1), thr.reshape(1, 1), Wm, bm.reshape(1, -1),
      Wo, bo.reshape(1, -1))

    out = out_slab[:_H * _WP].reshape(_H, _WP, C)[:, :_W, :]
    return jnp.transpose(out, (2, 0, 1))[None]
